# Initial kernel scaffold; baseline (speedup 1.0000x reference)
import jax
import jax.numpy as jnp
from jax.experimental import pallas as pl


def kernel(node_feats, node_attr, edge_src, edge_dst, edge_attr, edge_embedding, W_src, b_src, W_dst, W1, b1, W2, b2, W3, b3, W_sph, W_alpha, W_lin, W_sph2, W_val, attn_dot, W_out):
    raise NotImplementedError("write your pallas kernel here")


# speedup vs baseline: 13.3572x; 13.3572x over previous
import functools

import jax
import jax.numpy as jnp
from jax import lax
from jax.experimental import pallas as pl
from jax.experimental.pallas import tpu as pltpu
from jax.experimental.pallas import tpu_sc as plsc

_N, _E, _D, _DA, _DS, _DE, _H, _DH, _FC = 10000, 160000, 128, 16, 9, 16, 8, 16, 64
_NC, _NS = 2, 16
_NW = _NC * _NS
_CH = 128
_EPW = 5120
_EP = _NW * _EPW
_NCHUNK = _EPW // _CH
_NP = 10240
_RPS = _NP // _NS
_NP8 = _NP // 8
_RP8 = _NP8 // _NS

_BN = 1000
_BF = 80
_BE = 2048

_F32 = jnp.float32


def _slr(x):
    return 0.6 * x + 0.4 * x * (2.0 * jax.nn.sigmoid(x) - 1.0)


def _node_proj_body(nf, attr, wsrc, bsrc, wdst, w1b, w1c, osrc, odst, op):
    x = nf[...]
    osrc[...] = jnp.dot(x, wsrc[...], preferred_element_type=_F32) + bsrc[...]
    odst[...] = jnp.dot(x, wdst[...], preferred_element_type=_F32)
    a = attr[...]
    op[...] = jnp.concatenate(
        [jnp.dot(a, w1b[...], preferred_element_type=_F32),
         jnp.dot(a, w1c[...], preferred_element_type=_F32)], axis=-1)


def _node_proj(nf, attr, wsrc, bsrc, wdst, w1b, w1c):
    grid = (_N // _BN,)
    return pl.pallas_call(
        _node_proj_body,
        grid=grid,
        in_specs=[
            pl.BlockSpec((_BN, _D), lambda i: (i, 0)),
            pl.BlockSpec((_BN, _DA), lambda i: (i, 0)),
            pl.BlockSpec((_D, _D), lambda i: (0, 0)),
            pl.BlockSpec((1, _D), lambda i: (0, 0)),
            pl.BlockSpec((_D, _D), lambda i: (0, 0)),
            pl.BlockSpec((_DA, _FC), lambda i: (0, 0)),
            pl.BlockSpec((_DA, _FC), lambda i: (0, 0)),
        ],
        out_specs=[
            pl.BlockSpec((_BN, _D), lambda i: (i, 0)),
            pl.BlockSpec((_BN, _D), lambda i: (i, 0)),
            pl.BlockSpec((_BN, _D), lambda i: (i, 0)),
        ],
        out_shape=[
            jax.ShapeDtypeStruct((_N, _D), _F32),
            jax.ShapeDtypeStruct((_N, _D), _F32),
            jax.ShapeDtypeStruct((_N, _D), _F32),
        ],
    )(nf, attr, wsrc, bsrc, wdst, w1b, w1c)


def _gather_body(msrc, mdst, ptab, esrc, edst,
                 gs_out, gd_out, ps_out, pd_out,
                 idx_s, idx_d, gs_v, gd_v, ps_v, pd_v, sem):
    c = lax.axis_index("c")
    s = lax.axis_index("s")
    wid = s * _NC + c
    wbase = wid * _EPW

    def step(k, carry):
        base = wbase + k * _CH
        pltpu.sync_copy(esrc.at[pl.ds(base, _CH)], idx_s)
        pltpu.sync_copy(edst.at[pl.ds(base, _CH)], idx_d)
        cp1 = pltpu.async_copy(msrc.at[idx_s], gs_v, sem)
        cp2 = pltpu.async_copy(mdst.at[idx_d], gd_v, sem)
        cp3 = pltpu.async_copy(ptab.at[idx_s], ps_v, sem)
        cp4 = pltpu.async_copy(ptab.at[idx_d], pd_v, sem)
        cp1.wait()
        cp2.wait()
        cp3.wait()
        cp4.wait()
        pltpu.sync_copy(gs_v, gs_out.at[pl.ds(base, _CH)])
        pltpu.sync_copy(gd_v, gd_out.at[pl.ds(base, _CH)])
        pltpu.sync_copy(ps_v, ps_out.at[pl.ds(base, _CH)])
        pltpu.sync_copy(pd_v, pd_out.at[pl.ds(base, _CH)])
        return carry

    lax.fori_loop(0, _NCHUNK, step, 0)


def _sc_gather(msrc, mdst, ptab, esrc, edst):
    mesh = plsc.VectorSubcoreMesh(core_axis_name="c", subcore_axis_name="s",
                                  num_cores=_NC, num_subcores=_NS)
    f = pl.kernel(
        _gather_body,
        out_type=[
            jax.ShapeDtypeStruct((_EP, _D), _F32),
            jax.ShapeDtypeStruct((_EP, _D), _F32),
            jax.ShapeDtypeStruct((_EP, _D), _F32),
            jax.ShapeDtypeStruct((_EP, _D), _F32),
        ],
        mesh=mesh,
        scratch_types=[
            pltpu.VMEM((_CH,), jnp.int32),
            pltpu.VMEM((_CH,), jnp.int32),
            pltpu.VMEM((_CH, _D), _F32),
            pltpu.VMEM((_CH, _D), _F32),
            pltpu.VMEM((_CH, _D), _F32),
            pltpu.VMEM((_CH, _D), _F32),
            pltpu.SemaphoreType.DMA,
        ],
    )
    return f(msrc, mdst, ptab, esrc, edst)


def _edge_body(gs, gd, ps, pd, ee, ea, oh,
               w1a, b1, w2, b2, w3, b3,
               wsph, walpha, wlin, wsph2, wval, amat, rmat, tmat,
               numer, exps):
    m = gs[...] + gd[...]
    h = jnp.dot(ee[...], w1a[...], preferred_element_type=_F32)
    h = h + ps[:, :_FC] + pd[:, _FC:]
    h = jax.nn.silu(h + b1[...])
    h = jax.nn.silu(jnp.dot(h, w2[...], preferred_element_type=_F32) + b2[...])
    w = jnp.dot(h, w3[...], preferred_element_type=_F32) + b3[...]
    sph = jnp.dot(ea[...], wsph[...], preferred_element_type=_F32)
    msg = jax.nn.silu(m * w * sph)
    t = _slr(jnp.dot(msg, walpha[...], preferred_element_type=_F32))
    s16 = jnp.dot(t, amat[...], preferred_element_type=_F32)
    e16 = jnp.exp(s16)
    rowid = lax.broadcasted_iota(jnp.int32, e16.shape, 0) + pl.program_id(0) * _BE
    colid = lax.broadcasted_iota(jnp.int32, e16.shape, 1)
    e16 = jnp.where((rowid < _E) & (colid < _H), e16, 0.0)
    exps[...] = (jnp.dot(e16, tmat[...], preferred_element_type=_F32)
                 * jnp.dot(oh[...], rmat[...], preferred_element_type=_F32))
    v = jax.nn.silu(jnp.dot(msg, wlin[...], preferred_element_type=_F32))
    v = v * jnp.dot(ea[...], wsph2[...], preferred_element_type=_F32)
    v = jnp.dot(v, wval[...], preferred_element_type=_F32)
    numer[...] = v * jnp.dot(e16, rmat[...], preferred_element_type=_F32)


def _edge_stage(gs, gd, ps, pd, ee, ea, oh, weights):
    (w1a, b1, w2, b2, w3, b3,
     wsph, walpha, wlin, wsph2, wval, amat, rmat, tmat) = weights
    grid = (_EP // _BE,)
    full = lambda i: (0, 0)
    eblk = lambda i: (i, 0)
    return pl.pallas_call(
        _edge_body,
        grid=grid,
        in_specs=[
            pl.BlockSpec((_BE, _D), eblk),
            pl.BlockSpec((_BE, _D), eblk),
            pl.BlockSpec((_BE, _D), eblk),
            pl.BlockSpec((_BE, _D), eblk),
            pl.BlockSpec((_BE, _DE), eblk),
            pl.BlockSpec((_BE, 16), eblk),
            pl.BlockSpec((_BE, 16), eblk),
            pl.BlockSpec((_DE, _FC), full),
            pl.BlockSpec((1, _FC), full),
            pl.BlockSpec((_FC, _FC), full),
            pl.BlockSpec((1, _FC), full),
            pl.BlockSpec((_FC, _D), full),
            pl.BlockSpec((1, _D), full),
            pl.BlockSpec((16, _D), full),
            pl.BlockSpec((_D, _D), full),
            pl.BlockSpec((_D, _D), full),
            pl.BlockSpec((16, _D), full),
            pl.BlockSpec((_D, _D), full),
            pl.BlockSpec((_D, 16), full),
            pl.BlockSpec((16, _D), full),
            pl.BlockSpec((16, _D), full),
        ],
        out_specs=[
            pl.BlockSpec((_BE, _D), eblk),
            pl.BlockSpec((_BE, _D), eblk),
        ],
        out_shape=[
            jax.ShapeDtypeStruct((_EP, _D), _F32),
            jax.ShapeDtypeStruct((_EP, _D), _F32),
        ],
    )(gs, gd, ps, pd, ee, ea, oh, w1a, b1, w2, b2, w3, b3,
      wsph, walpha, wlin, wsph2, wval, amat, rmat, tmat)


def _scatter_body(numer, exps, edst, edst8, zn,
                  np_out, ep_out,
                  acc_n, acc_e, idx_v, nbuf, ebuf):
    c = lax.axis_index("c")
    s = lax.axis_index("s")
    rbase = s * _RPS
    r8base = s * _RP8
    pltpu.sync_copy(zn.at[pl.ds(0, _CH)], nbuf)
    for j in range(_RPS // _CH):
        pltpu.sync_copy(nbuf, acc_n.at[pl.ds(rbase + j * _CH, _CH)])
    pltpu.sync_copy(nbuf.at[pl.ds(0, _RP8)], acc_e.at[pl.ds(r8base, _RP8)])
    plsc.subcore_barrier()
    wbase = (c * _NS + s) * _EPW

    def step(k, carry):
        base = wbase + k * _CH
        row_n = idx_v.at[0]
        row_e = idx_v.at[1]
        pltpu.sync_copy(edst.at[pl.ds(base, _CH)], row_n)
        pltpu.sync_copy(edst8.at[pl.ds(base, _CH)], row_e)
        pltpu.sync_copy(numer.at[pl.ds(base, _CH)], nbuf)
        pltpu.sync_copy(exps.at[pl.ds(base, _CH)], ebuf)
        pltpu.sync_copy(nbuf, acc_n.at[row_n], add=True)
        pltpu.sync_copy(ebuf, acc_e.at[row_e], add=True)
        return carry

    lax.fori_loop(0, _NCHUNK, step, 0)
    plsc.subcore_barrier()
    obase = c * _NP + rbase
    for j in range(_RPS // _CH):
        pltpu.sync_copy(acc_n.at[pl.ds(rbase + j * _CH, _CH)], nbuf)
        pltpu.sync_copy(nbuf, np_out.at[pl.ds(obase + j * _CH, _CH)])
    pltpu.sync_copy(acc_e.at[pl.ds(r8base, _RP8)], ebuf.at[pl.ds(0, _RP8)])
    pltpu.sync_copy(ebuf.at[pl.ds(0, _RP8)],
                    ep_out.at[pl.ds(c * _NP8 + r8base, _RP8)])


def _sc_scatter(numer, exps, edst, edst8, zn):
    mesh = plsc.VectorSubcoreMesh(core_axis_name="c", subcore_axis_name="s",
                                  num_cores=_NC, num_subcores=_NS)
    f = pl.kernel(
        _scatter_body,
        out_type=[
            jax.ShapeDtypeStruct((_NC * _NP, _D), _F32),
            jax.ShapeDtypeStruct((_NC * _NP8, _D), _F32),
        ],
        mesh=mesh,
        scratch_types=[
            pltpu.VMEM_SHARED((_NP, _D), _F32),
            pltpu.VMEM_SHARED((_NP8, _D), _F32),
            pltpu.VMEM((2, _CH), jnp.int32),
            pltpu.VMEM((_CH, _D), _F32),
            pltpu.VMEM((_CH, _D), _F32),
        ],
    )
    return f(numer, exps, edst, edst8, zn)


def _final_body(n0, n1, e0, e1, wout, rmat, o):
    numer = n0[...] + n1[...]
    den = jnp.dot(e0[...] + e1[...], rmat[...], preferred_element_type=_F32)
    o[...] = jnp.dot(numer / (den + 1e-16), wout[...], preferred_element_type=_F32)


def _final_stage(np2, ep2, wout, rmat):
    grid = (_N // _BF,)
    off = _NP // _BF
    return pl.pallas_call(
        _final_body,
        grid=grid,
        in_specs=[
            pl.BlockSpec((_BF, _D), lambda i: (i, 0)),
            pl.BlockSpec((_BF, _D), lambda i: (i + off, 0)),
            pl.BlockSpec((_BF, 16), lambda i: (i, 0)),
            pl.BlockSpec((_BF, 16), lambda i: (i + off, 0)),
            pl.BlockSpec((_D, _D), lambda i: (0, 0)),
            pl.BlockSpec((16, _D), lambda i: (0, 0)),
        ],
        out_specs=pl.BlockSpec((_BF, _D), lambda i: (i, 0)),
        out_shape=jax.ShapeDtypeStruct((_N, _D), _F32),
    )(np2, np2, ep2, ep2, wout, rmat)


def kernel(node_feats, node_attr, edge_src, edge_dst, edge_attr, edge_embedding,
           W_src, b_src, W_dst, W1, b1, W2, b2, W3, b3,
           W_sph, W_alpha, W_lin, W_sph2, W_val, attn_dot, W_out):
    pad = _EP - _E
    esrc_p = jnp.concatenate([edge_src.astype(jnp.int32),
                              jnp.zeros((pad,), jnp.int32)])
    edst_p = jnp.concatenate([edge_dst.astype(jnp.int32),
                              jnp.zeros((pad,), jnp.int32)])
    ea16 = jnp.zeros((_EP, 16), _F32).at[:_E, :_DS].set(edge_attr)
    ee_p = jnp.zeros((_EP, _DE), _F32).at[:_E].set(edge_embedding)
    w1a, w1b, w1c = W1[:_DE], W1[_DE:_DE + _DA], W1[_DE + _DA:]
    wsph16 = jnp.zeros((16, _D), _F32).at[:_DS].set(W_sph)
    wsph2_16 = jnp.zeros((16, _D), _F32).at[:_DS].set(W_sph2)
    amat = (attn_dot[:, :, None] * jnp.eye(_H, dtype=_F32)[:, None, :])
    amat = amat.reshape(_H * _DH, _H)
    amat = jnp.pad(amat, ((0, 0), (0, 16 - _H)))
    rmat = (jnp.eye(_H, dtype=_F32)[:, :, None]
            * jnp.ones((_DH,), _F32)).reshape(_H, _H * _DH)
    rmat = jnp.pad(rmat, ((0, 16 - _H), (0, 0)))
    tmat = jnp.concatenate([jnp.eye(16, dtype=_F32)] * _H, axis=1)
    edst8_p = edst_p // 8
    oh = (jnp.arange(16, dtype=jnp.int32)[None, :]
          == (edst_p % 8)[:, None]).astype(_F32)
    zn = jnp.zeros((_NP, _D), _F32)

    msrc, mdst, ptab = _node_proj(node_feats, node_attr, W_src,
                                  b_src.reshape(1, _D), W_dst, w1b, w1c)
    gs, gd, ps, pd = _sc_gather(msrc, mdst, ptab, esrc_p, edst_p)
    numer, exps = _edge_stage(
        gs, gd, ps, pd, ee_p, ea16, oh,
        (w1a, b1.reshape(1, _FC), W2, b2.reshape(1, _FC),
         W3, b3.reshape(1, _D), wsph16, W_alpha, W_lin, wsph2_16, W_val,
         amat, rmat, tmat))
    np2, ep8 = _sc_scatter(numer, exps, edst_p, edst8_p, zn)
    ep2 = ep8.reshape(_NC * _NP, 16)
    return _final_stage(np2, ep2, W_out, rmat)


# speedup vs baseline: 15.6328x; 1.1704x over previous
import functools

import jax
import jax.numpy as jnp
from jax import lax
from jax.experimental import pallas as pl
from jax.experimental.pallas import tpu as pltpu
from jax.experimental.pallas import tpu_sc as plsc

_N, _E, _D, _DA, _DS, _DE, _H, _DH, _FC = 10000, 160000, 128, 16, 9, 16, 8, 16, 64
_NC, _NS = 2, 16
_NW = _NC * _NS
_CH = 128
_EPW = 5120
_EP = _NW * _EPW
_NCHUNK = _EPW // _CH
_NP = 10240
_RPS = _NP // _NS
_NP8 = _NP // 8
_RP8 = _NP8 // _NS

_BN = 1000
_BF = 80
_BE = 2048

_F32 = jnp.float32


def _slr(x):
    return 0.6 * x + 0.4 * x * (2.0 * jax.nn.sigmoid(x) - 1.0)


def _node_proj_body(nf, attr, wsrc, bsrc, wdst, w1b, w1c, osrc, odst, op):
    x = nf[...]
    osrc[...] = jnp.dot(x, wsrc[...], preferred_element_type=_F32) + bsrc[...]
    odst[...] = jnp.dot(x, wdst[...], preferred_element_type=_F32)
    a = attr[...]
    op[...] = jnp.concatenate(
        [jnp.dot(a, w1b[...], preferred_element_type=_F32),
         jnp.dot(a, w1c[...], preferred_element_type=_F32)], axis=-1)


def _node_proj(nf, attr, wsrc, bsrc, wdst, w1b, w1c):
    grid = (_N // _BN,)
    return pl.pallas_call(
        _node_proj_body,
        grid=grid,
        in_specs=[
            pl.BlockSpec((_BN, _D), lambda i: (i, 0)),
            pl.BlockSpec((_BN, _DA), lambda i: (i, 0)),
            pl.BlockSpec((_D, _D), lambda i: (0, 0)),
            pl.BlockSpec((1, _D), lambda i: (0, 0)),
            pl.BlockSpec((_D, _D), lambda i: (0, 0)),
            pl.BlockSpec((_DA, _FC), lambda i: (0, 0)),
            pl.BlockSpec((_DA, _FC), lambda i: (0, 0)),
        ],
        out_specs=[
            pl.BlockSpec((_BN, _D), lambda i: (i, 0)),
            pl.BlockSpec((_BN, _D), lambda i: (i, 0)),
            pl.BlockSpec((_BN, _D), lambda i: (i, 0)),
        ],
        out_shape=[
            jax.ShapeDtypeStruct((_N, _D), _F32),
            jax.ShapeDtypeStruct((_N, _D), _F32),
            jax.ShapeDtypeStruct((_N, _D), _F32),
        ],
    )(nf, attr, wsrc, bsrc, wdst, w1b, w1c)


_GC = 64
_GCHUNK = _EPW // _GC


def _gather_body(msrc, mdst, ptab, esrc2, edst2,
                 gs_out, gd_out, ps_out, pd_out,
                 idx_s, idx_d,
                 gsa, gda, psa, pda, gsb, gdb, psb, pdb, sema, semb):
    c = lax.axis_index("c")
    s = lax.axis_index("s")
    wid = s * _NC + c
    wbase = wid * _EPW
    pltpu.sync_copy(esrc2.at[pl.ds(wid * _GCHUNK, _GCHUNK)], idx_s)
    pltpu.sync_copy(edst2.at[pl.ds(wid * _GCHUNK, _GCHUNK)], idx_d)

    def fire(k, bufs, sem):
        gs_v, gd_v, ps_v, pd_v = bufs
        pltpu.async_copy(msrc.at[idx_s.at[k]], gs_v, sem)
        pltpu.async_copy(mdst.at[idx_d.at[k]], gd_v, sem)
        pltpu.async_copy(ptab.at[idx_s.at[k]], ps_v, sem)
        pltpu.async_copy(ptab.at[idx_d.at[k]], pd_v, sem)

    def drain_write(k, bufs, sem):
        gs_v, gd_v, ps_v, pd_v = bufs
        for buf in bufs:
            pltpu.make_async_copy(msrc.at[idx_s.at[0]], buf, sem).wait()
        base = wbase + k * _GC
        pltpu.sync_copy(gs_v, gs_out.at[pl.ds(base, _GC)])
        pltpu.sync_copy(gd_v, gd_out.at[pl.ds(base, _GC)])
        pltpu.sync_copy(ps_v, ps_out.at[pl.ds(base, _GC)])
        pltpu.sync_copy(pd_v, pd_out.at[pl.ds(base, _GC)])

    bufa = (gsa, gda, psa, pda)
    bufb = (gsb, gdb, psb, pdb)
    fire(0, bufa, sema)
    fire(1, bufb, semb)

    def step(j, carry):
        k0 = 2 * j
        drain_write(k0, bufa, sema)
        fire(k0 + 2, bufa, sema)
        drain_write(k0 + 1, bufb, semb)
        fire(k0 + 3, bufb, semb)
        return carry

    lax.fori_loop(0, _GCHUNK // 2 - 1, step, 0)
    drain_write(_GCHUNK - 2, bufa, sema)
    drain_write(_GCHUNK - 1, bufb, semb)


def _sc_gather(msrc, mdst, ptab, esrc2, edst2):
    mesh = plsc.VectorSubcoreMesh(core_axis_name="c", subcore_axis_name="s",
                                  num_cores=_NC, num_subcores=_NS)
    buf = lambda: pltpu.VMEM((_GC, _D), _F32)
    f = pl.kernel(
        _gather_body,
        out_type=[
            jax.ShapeDtypeStruct((_EP, _D), _F32),
            jax.ShapeDtypeStruct((_EP, _D), _F32),
            jax.ShapeDtypeStruct((_EP, _D), _F32),
            jax.ShapeDtypeStruct((_EP, _D), _F32),
        ],
        mesh=mesh,
        scratch_types=[
            pltpu.VMEM((_GCHUNK, _GC), jnp.int32),
            pltpu.VMEM((_GCHUNK, _GC), jnp.int32),
            buf(), buf(), buf(), buf(),
            buf(), buf(), buf(), buf(),
            pltpu.SemaphoreType.DMA,
            pltpu.SemaphoreType.DMA,
        ],
    )
    return f(msrc, mdst, ptab, esrc2, edst2)


def _edge_body(gs, gd, ps, pd, ee, ea, oh,
               w1a, b1, w2, b2, w3, b3,
               wsph, walpha, wlin, wsph2, wval, amat, rmat, tmat,
               numer, exps):
    m = gs[...] + gd[...]
    h = jnp.dot(ee[...], w1a[...], preferred_element_type=_F32)
    h = h + ps[:, :_FC] + pd[:, _FC:]
    h = jax.nn.silu(h + b1[...])
    h = jax.nn.silu(jnp.dot(h, w2[...], preferred_element_type=_F32) + b2[...])
    w = jnp.dot(h, w3[...], preferred_element_type=_F32) + b3[...]
    sph = jnp.dot(ea[...], wsph[...], preferred_element_type=_F32)
    msg = jax.nn.silu(m * w * sph)
    t = _slr(jnp.dot(msg, walpha[...], preferred_element_type=_F32))
    s16 = jnp.dot(t, amat[...], preferred_element_type=_F32)
    e16 = jnp.exp(s16)
    rowid = lax.broadcasted_iota(jnp.int32, e16.shape, 0) + pl.program_id(0) * _BE
    colid = lax.broadcasted_iota(jnp.int32, e16.shape, 1)
    e16 = jnp.where((rowid < _E) & (colid < _H), e16, 0.0)
    exps[...] = (jnp.dot(e16, tmat[...], preferred_element_type=_F32)
                 * jnp.dot(oh[...], rmat[...], preferred_element_type=_F32))
    v = jax.nn.silu(jnp.dot(msg, wlin[...], preferred_element_type=_F32))
    v = v * jnp.dot(ea[...], wsph2[...], preferred_element_type=_F32)
    v = jnp.dot(v, wval[...], preferred_element_type=_F32)
    numer[...] = v * jnp.dot(e16, rmat[...], preferred_element_type=_F32)


def _edge_stage(gs, gd, ps, pd, ee, ea, oh, weights):
    (w1a, b1, w2, b2, w3, b3,
     wsph, walpha, wlin, wsph2, wval, amat, rmat, tmat) = weights
    grid = (_EP // _BE,)
    full = lambda i: (0, 0)
    eblk = lambda i: (i, 0)
    return pl.pallas_call(
        _edge_body,
        grid=grid,
        in_specs=[
            pl.BlockSpec((_BE, _D), eblk),
            pl.BlockSpec((_BE, _D), eblk),
            pl.BlockSpec((_BE, _D), eblk),
            pl.BlockSpec((_BE, _D), eblk),
            pl.BlockSpec((_BE, _DE), eblk),
            pl.BlockSpec((_BE, 16), eblk),
            pl.BlockSpec((_BE, 16), eblk),
            pl.BlockSpec((_DE, _FC), full),
            pl.BlockSpec((1, _FC), full),
            pl.BlockSpec((_FC, _FC), full),
            pl.BlockSpec((1, _FC), full),
            pl.BlockSpec((_FC, _D), full),
            pl.BlockSpec((1, _D), full),
            pl.BlockSpec((16, _D), full),
            pl.BlockSpec((_D, _D), full),
            pl.BlockSpec((_D, _D), full),
            pl.BlockSpec((16, _D), full),
            pl.BlockSpec((_D, _D), full),
            pl.BlockSpec((_D, 16), full),
            pl.BlockSpec((16, _D), full),
            pl.BlockSpec((16, _D), full),
        ],
        out_specs=[
            pl.BlockSpec((_BE, _D), eblk),
            pl.BlockSpec((_BE, _D), eblk),
        ],
        out_shape=[
            jax.ShapeDtypeStruct((_EP, _D), _F32),
            jax.ShapeDtypeStruct((_EP, _D), _F32),
        ],
    )(gs, gd, ps, pd, ee, ea, oh, w1a, b1, w2, b2, w3, b3,
      wsph, walpha, wlin, wsph2, wval, amat, rmat, tmat)


def _scatter_body(numer, exps, edst, edst8, zn,
                  np_out, ep_out,
                  acc_n, acc_e, idx_v, nbuf, ebuf):
    c = lax.axis_index("c")
    s = lax.axis_index("s")
    rbase = s * _RPS
    r8base = s * _RP8
    pltpu.sync_copy(zn.at[pl.ds(0, _CH)], nbuf)
    for j in range(_RPS // _CH):
        pltpu.sync_copy(nbuf, acc_n.at[pl.ds(rbase + j * _CH, _CH)])
    pltpu.sync_copy(nbuf.at[pl.ds(0, _RP8)], acc_e.at[pl.ds(r8base, _RP8)])
    plsc.subcore_barrier()
    wbase = (c * _NS + s) * _EPW

    def step(k, carry):
        base = wbase + k * _CH
        row_n = idx_v.at[0]
        row_e = idx_v.at[1]
        pltpu.sync_copy(edst.at[pl.ds(base, _CH)], row_n)
        pltpu.sync_copy(edst8.at[pl.ds(base, _CH)], row_e)
        pltpu.sync_copy(numer.at[pl.ds(base, _CH)], nbuf)
        pltpu.sync_copy(exps.at[pl.ds(base, _CH)], ebuf)
        pltpu.sync_copy(nbuf, acc_n.at[row_n], add=True)
        pltpu.sync_copy(ebuf, acc_e.at[row_e], add=True)
        return carry

    lax.fori_loop(0, _NCHUNK, step, 0)
    plsc.subcore_barrier()
    obase = c * _NP + rbase
    for j in range(_RPS // _CH):
        pltpu.sync_copy(acc_n.at[pl.ds(rbase + j * _CH, _CH)], nbuf)
        pltpu.sync_copy(nbuf, np_out.at[pl.ds(obase + j * _CH, _CH)])
    pltpu.sync_copy(acc_e.at[pl.ds(r8base, _RP8)], ebuf.at[pl.ds(0, _RP8)])
    pltpu.sync_copy(ebuf.at[pl.ds(0, _RP8)],
                    ep_out.at[pl.ds(c * _NP8 + r8base, _RP8)])


def _sc_scatter(numer, exps, edst, edst8, zn):
    mesh = plsc.VectorSubcoreMesh(core_axis_name="c", subcore_axis_name="s",
                                  num_cores=_NC, num_subcores=_NS)
    f = pl.kernel(
        _scatter_body,
        out_type=[
            jax.ShapeDtypeStruct((_NC * _NP, _D), _F32),
            jax.ShapeDtypeStruct((_NC * _NP8, _D), _F32),
        ],
        mesh=mesh,
        scratch_types=[
            pltpu.VMEM_SHARED((_NP, _D), _F32),
            pltpu.VMEM_SHARED((_NP8, _D), _F32),
            pltpu.VMEM((2, _CH), jnp.int32),
            pltpu.VMEM((_CH, _D), _F32),
            pltpu.VMEM((_CH, _D), _F32),
        ],
    )
    return f(numer, exps, edst, edst8, zn)


def _final_body(n0, n1, e0, e1, wout, rmat, o):
    numer = n0[...] + n1[...]
    den = jnp.dot(e0[...] + e1[...], rmat[...], preferred_element_type=_F32)
    o[...] = jnp.dot(numer / (den + 1e-16), wout[...], preferred_element_type=_F32)


def _final_stage(np2, ep2, wout, rmat):
    grid = (_N // _BF,)
    off = _NP // _BF
    return pl.pallas_call(
        _final_body,
        grid=grid,
        in_specs=[
            pl.BlockSpec((_BF, _D), lambda i: (i, 0)),
            pl.BlockSpec((_BF, _D), lambda i: (i + off, 0)),
            pl.BlockSpec((_BF, 16), lambda i: (i, 0)),
            pl.BlockSpec((_BF, 16), lambda i: (i + off, 0)),
            pl.BlockSpec((_D, _D), lambda i: (0, 0)),
            pl.BlockSpec((16, _D), lambda i: (0, 0)),
        ],
        out_specs=pl.BlockSpec((_BF, _D), lambda i: (i, 0)),
        out_shape=jax.ShapeDtypeStruct((_N, _D), _F32),
    )(np2, np2, ep2, ep2, wout, rmat)


def kernel(node_feats, node_attr, edge_src, edge_dst, edge_attr, edge_embedding,
           W_src, b_src, W_dst, W1, b1, W2, b2, W3, b3,
           W_sph, W_alpha, W_lin, W_sph2, W_val, attn_dot, W_out):
    pad = _EP - _E
    esrc_p = jnp.concatenate([edge_src.astype(jnp.int32),
                              jnp.zeros((pad,), jnp.int32)])
    edst_p = jnp.concatenate([edge_dst.astype(jnp.int32),
                              jnp.zeros((pad,), jnp.int32)])
    ea16 = jnp.zeros((_EP, 16), _F32).at[:_E, :_DS].set(edge_attr)
    ee_p = jnp.zeros((_EP, _DE), _F32).at[:_E].set(edge_embedding)
    w1a, w1b, w1c = W1[:_DE], W1[_DE:_DE + _DA], W1[_DE + _DA:]
    wsph16 = jnp.zeros((16, _D), _F32).at[:_DS].set(W_sph)
    wsph2_16 = jnp.zeros((16, _D), _F32).at[:_DS].set(W_sph2)
    amat = (attn_dot[:, :, None] * jnp.eye(_H, dtype=_F32)[:, None, :])
    amat = amat.reshape(_H * _DH, _H)
    amat = jnp.pad(amat, ((0, 0), (0, 16 - _H)))
    rmat = (jnp.eye(_H, dtype=_F32)[:, :, None]
            * jnp.ones((_DH,), _F32)).reshape(_H, _H * _DH)
    rmat = jnp.pad(rmat, ((0, 16 - _H), (0, 0)))
    tmat = jnp.concatenate([jnp.eye(16, dtype=_F32)] * _H, axis=1)
    edst8_p = edst_p // 8
    oh = (jnp.arange(16, dtype=jnp.int32)[None, :]
          == (edst_p % 8)[:, None]).astype(_F32)
    zn = jnp.zeros((_NP, _D), _F32)

    msrc, mdst, ptab = _node_proj(node_feats, node_attr, W_src,
                                  b_src.reshape(1, _D), W_dst, w1b, w1c)
    gs, gd, ps, pd = _sc_gather(msrc, mdst, ptab,
                                esrc_p.reshape(_EP // _GC, _GC),
                                edst_p.reshape(_EP // _GC, _GC))
    numer, exps = _edge_stage(
        gs, gd, ps, pd, ee_p, ea16, oh,
        (w1a, b1.reshape(1, _FC), W2, b2.reshape(1, _FC),
         W3, b3.reshape(1, _D), wsph16, W_alpha, W_lin, wsph2_16, W_val,
         amat, rmat, tmat))
    np2, ep8 = _sc_scatter(numer, exps, edst_p, edst8_p, zn)
    ep2 = ep8.reshape(_NC * _NP, 16)
    return _final_stage(np2, ep2, W_out, rmat)


# speedup vs baseline: 23.2398x; 1.4866x over previous
import functools

import jax
import jax.numpy as jnp
from jax import lax
from jax.experimental import pallas as pl
from jax.experimental.pallas import tpu as pltpu
from jax.experimental.pallas import tpu_sc as plsc

_N, _E, _D, _DA, _DS, _DE, _H, _DH, _FC = 10000, 160000, 128, 16, 9, 16, 8, 16, 64
_NC, _NS = 2, 16
_NW = _NC * _NS
_CH = 128
_EPW = 5120
_EP = _NW * _EPW
_NCHUNK = _EPW // _CH
_NP = 10240
_RPS = _NP // _NS
_NP8 = _NP // 8
_RP8 = _NP8 // _NS

_BN = 1000
_BF = 80
_BE = 2048

_F32 = jnp.float32


def _slr(x):
    return 0.6 * x + 0.4 * x * (2.0 * jax.nn.sigmoid(x) - 1.0)


_BF16 = jnp.bfloat16


def _pack16(lo_f32, hi_f32):
    lo = lax.bitcast_convert_type(lo_f32.astype(_BF16), jnp.uint16)
    hi = lax.bitcast_convert_type(hi_f32.astype(_BF16), jnp.uint16)
    w = lo.astype(jnp.uint32) | (hi.astype(jnp.uint32) << 16)
    return lax.bitcast_convert_type(w, jnp.int32)


def _node_proj_body(nf, attr, wsrc, bsrc, wdst, w1b, w1c, t1, t2):
    x = nf[...]
    a = attr[...]
    ms = jnp.dot(x, wsrc[...], preferred_element_type=_F32) + bsrc[...]
    md = jnp.dot(x, wdst[...], preferred_element_type=_F32)
    p = jnp.concatenate(
        [jnp.dot(a, w1b[...], preferred_element_type=_F32),
         jnp.dot(a, w1c[...], preferred_element_type=_F32)], axis=-1)
    t1[...] = _pack16(ms, p)
    t2[...] = _pack16(md, p)


def _node_proj(nf, attr, wsrc, bsrc, wdst, w1b, w1c):
    grid = (_N // _BN,)
    return pl.pallas_call(
        _node_proj_body,
        grid=grid,
        in_specs=[
            pl.BlockSpec((_BN, _D), lambda i: (i, 0)),
            pl.BlockSpec((_BN, _DA), lambda i: (i, 0)),
            pl.BlockSpec((_D, _D), lambda i: (0, 0)),
            pl.BlockSpec((1, _D), lambda i: (0, 0)),
            pl.BlockSpec((_D, _D), lambda i: (0, 0)),
            pl.BlockSpec((_DA, _FC), lambda i: (0, 0)),
            pl.BlockSpec((_DA, _FC), lambda i: (0, 0)),
        ],
        out_specs=[
            pl.BlockSpec((_BN, _D), lambda i: (i, 0)),
            pl.BlockSpec((_BN, _D), lambda i: (i, 0)),
        ],
        out_shape=[
            jax.ShapeDtypeStruct((_N, _D), jnp.int32),
            jax.ShapeDtypeStruct((_N, _D), jnp.int32),
        ],
    )(nf, attr, wsrc, bsrc, wdst, w1b, w1c)


_GC = 64
_GCHUNK = _EPW // _GC
_SC = 64
_SCHUNK = _EPW // _SC


def _gather_body(t1, t2, esrc2, edst2,
                 o1_out, o2_out,
                 idx_s, idx_d,
                 b1a, b2a, b1b, b2b, sema, semb):
    c = lax.axis_index("c")
    s = lax.axis_index("s")
    wid = s * _NC + c
    wbase = wid * _EPW
    pltpu.sync_copy(esrc2.at[pl.ds(wid * _GCHUNK, _GCHUNK)], idx_s)
    pltpu.sync_copy(edst2.at[pl.ds(wid * _GCHUNK, _GCHUNK)], idx_d)

    def fire(k, bufs, sem):
        v1, v2 = bufs
        pltpu.async_copy(t1.at[idx_s.at[k]], v1, sem)
        pltpu.async_copy(t2.at[idx_d.at[k]], v2, sem)

    def drain_write(k, bufs, sem):
        v1, v2 = bufs
        for buf in bufs:
            pltpu.make_async_copy(t1.at[idx_s.at[0]], buf, sem).wait()
        base = wbase + k * _GC
        pltpu.sync_copy(v1, o1_out.at[pl.ds(base, _GC)])
        pltpu.sync_copy(v2, o2_out.at[pl.ds(base, _GC)])

    bufa = (b1a, b2a)
    bufb = (b1b, b2b)
    fire(0, bufa, sema)
    fire(1, bufb, semb)

    def step(j, carry):
        k0 = 2 * j
        drain_write(k0, bufa, sema)
        fire(k0 + 2, bufa, sema)
        drain_write(k0 + 1, bufb, semb)
        fire(k0 + 3, bufb, semb)
        return carry

    lax.fori_loop(0, _GCHUNK // 2 - 1, step, 0)
    drain_write(_GCHUNK - 2, bufa, sema)
    drain_write(_GCHUNK - 1, bufb, semb)


def _sc_gather(t1, t2, esrc2, edst2):
    mesh = plsc.VectorSubcoreMesh(core_axis_name="c", subcore_axis_name="s",
                                  num_cores=_NC, num_subcores=_NS)
    buf = lambda: pltpu.VMEM((_GC, _D), jnp.int32)
    f = pl.kernel(
        _gather_body,
        out_type=[
            jax.ShapeDtypeStruct((_EP, _D), jnp.int32),
            jax.ShapeDtypeStruct((_EP, _D), jnp.int32),
        ],
        mesh=mesh,
        scratch_types=[
            pltpu.VMEM((_GCHUNK, _GC), jnp.int32),
            pltpu.VMEM((_GCHUNK, _GC), jnp.int32),
            buf(), buf(),
            buf(), buf(),
            pltpu.SemaphoreType.DMA,
            pltpu.SemaphoreType.DMA,
        ],
    )
    return f(t1, t2, esrc2, edst2)


def _edge_body(t1, t2, ee, ea, oh,
               w1a, b1, w2, b2, w3, b3,
               wsph, walpha, wlin, wsph2, wval, amat, rmat, tmat,
               numer, exps):
    x1 = lax.bitcast_convert_type(t1[...], jnp.uint32)
    x2 = lax.bitcast_convert_type(t2[...], jnp.uint32)
    ms1 = lax.bitcast_convert_type(x1 << 16, _F32)
    ms2 = lax.bitcast_convert_type(x2 << 16, _F32)
    pp1 = lax.bitcast_convert_type(x1 & jnp.uint32(0xFFFF0000), _F32)
    pp2 = lax.bitcast_convert_type(x2 & jnp.uint32(0xFFFF0000), _F32)
    m = ms1 + ms2
    h = jnp.dot(ee[...], w1a[...], preferred_element_type=_F32)
    h = h + pp1[:, :_FC] + pp2[:, _FC:]
    h = jax.nn.silu(h + b1[...])
    h = jax.nn.silu(jnp.dot(h, w2[...], preferred_element_type=_F32) + b2[...])
    w = jnp.dot(h, w3[...], preferred_element_type=_F32) + b3[...]
    sph = jnp.dot(ea[...], wsph[...], preferred_element_type=_F32)
    msg = jax.nn.silu(m * w * sph)
    t = _slr(jnp.dot(msg, walpha[...], preferred_element_type=_F32))
    s16 = jnp.dot(t, amat[...], preferred_element_type=_F32)
    e16 = jnp.exp(s16)
    rowid = lax.broadcasted_iota(jnp.int32, e16.shape, 0) + pl.program_id(0) * _BE
    colid = lax.broadcasted_iota(jnp.int32, e16.shape, 1)
    e16 = jnp.where((rowid < _E) & (colid < _H), e16, 0.0)
    exps[...] = (jnp.dot(e16, tmat[...], preferred_element_type=_F32)
                 * jnp.dot(oh[...], rmat[...], preferred_element_type=_F32))
    v = jax.nn.silu(jnp.dot(msg, wlin[...], preferred_element_type=_F32))
    v = v * jnp.dot(ea[...], wsph2[...], preferred_element_type=_F32)
    v = jnp.dot(v, wval[...], preferred_element_type=_F32)
    numer[...] = v * jnp.dot(e16, rmat[...], preferred_element_type=_F32)


def _edge_stage(t1e, t2e, ee, ea, oh, weights):
    (w1a, b1, w2, b2, w3, b3,
     wsph, walpha, wlin, wsph2, wval, amat, rmat, tmat) = weights
    grid = (_EP // _BE,)
    full = lambda i: (0, 0)
    eblk = lambda i: (i, 0)
    return pl.pallas_call(
        _edge_body,
        grid=grid,
        in_specs=[
            pl.BlockSpec((_BE, _D), eblk),
            pl.BlockSpec((_BE, _D), eblk),
            pl.BlockSpec((_BE, _DE), eblk),
            pl.BlockSpec((_BE, 16), eblk),
            pl.BlockSpec((_BE, 16), eblk),
            pl.BlockSpec((_DE, _FC), full),
            pl.BlockSpec((1, _FC), full),
            pl.BlockSpec((_FC, _FC), full),
            pl.BlockSpec((1, _FC), full),
            pl.BlockSpec((_FC, _D), full),
            pl.BlockSpec((1, _D), full),
            pl.BlockSpec((16, _D), full),
            pl.BlockSpec((_D, _D), full),
            pl.BlockSpec((_D, _D), full),
            pl.BlockSpec((16, _D), full),
            pl.BlockSpec((_D, _D), full),
            pl.BlockSpec((_D, 16), full),
            pl.BlockSpec((16, _D), full),
            pl.BlockSpec((16, _D), full),
        ],
        out_specs=[
            pl.BlockSpec((_BE, _D), eblk),
            pl.BlockSpec((_BE, _D), eblk),
        ],
        out_shape=[
            jax.ShapeDtypeStruct((_EP, _D), _F32),
            jax.ShapeDtypeStruct((_EP, _D), _F32),
        ],
    )(t1e, t2e, ee, ea, oh, w1a, b1, w2, b2, w3, b3,
      wsph, walpha, wlin, wsph2, wval, amat, rmat, tmat)


def _scatter_body(numer, exps, edst2, zn,
                  np_out, ep_out,
                  acc_n, acc_e, ixa, ixb, i8a, i8b,
                  nba, eba, nbb, ebb, sema, semb):
    c = lax.axis_index("c")
    s = lax.axis_index("s")
    rbase = s * _RPS
    r8base = s * _RP8
    pltpu.sync_copy(zn.at[pl.ds(0, _SC)], nba)
    for j in range(_RPS // _SC):
        pltpu.sync_copy(nba, acc_n.at[pl.ds(rbase + j * _SC, _SC)])
    pltpu.sync_copy(nba, acc_e.at[pl.ds(r8base, _SC)])
    pltpu.sync_copy(nba.at[pl.ds(0, _RP8 - _SC)],
                    acc_e.at[pl.ds(r8base + _SC, _RP8 - _SC)])
    wid = c * _NS + s
    wbase = wid * _EPW
    plsc.subcore_barrier()

    def fire(k, bufs, sem):
        nb, eb, ix, i8 = bufs
        base = wbase + k * _SC
        pltpu.async_copy(numer.at[pl.ds(base, _SC)], nb, sem)
        pltpu.async_copy(exps.at[pl.ds(base, _SC)], eb, sem)
        pltpu.async_copy(edst2.at[pl.ds(wid * _SCHUNK + k, 1)], ix, sem)

    def drain_scatter(k, bufs, sem):
        nb, eb, ix, i8 = bufs
        pltpu.make_async_copy(numer.at[pl.ds(0, _SC)], nb, sem).wait()
        pltpu.make_async_copy(numer.at[pl.ds(0, _SC)], eb, sem).wait()
        pltpu.make_async_copy(edst2.at[pl.ds(0, 1)], ix, sem).wait()
        row = ix.at[0]
        r8 = i8.at[0]
        for i in range(_SC // 16):
            r8[pl.ds(i * 16, 16)] = row[pl.ds(i * 16, 16)] >> 3
        pltpu.sync_copy(nb, acc_n.at[row], add=True)
        pltpu.sync_copy(eb, acc_e.at[r8], add=True)

    bufa = (nba, eba, ixa, i8a)
    bufb = (nbb, ebb, ixb, i8b)
    fire(0, bufa, sema)
    fire(1, bufb, semb)

    def step(j, carry):
        k0 = 2 * j
        drain_scatter(k0, bufa, sema)
        fire(k0 + 2, bufa, sema)
        drain_scatter(k0 + 1, bufb, semb)
        fire(k0 + 3, bufb, semb)
        return carry

    lax.fori_loop(0, _SCHUNK // 2 - 1, step, 0)
    drain_scatter(_SCHUNK - 2, bufa, sema)
    drain_scatter(_SCHUNK - 1, bufb, semb)
    plsc.subcore_barrier()
    obase = c * _NP + rbase
    for j in range(_RPS // _SC):
        pltpu.sync_copy(acc_n.at[pl.ds(rbase + j * _SC, _SC)], nba)
        pltpu.sync_copy(nba, np_out.at[pl.ds(obase + j * _SC, _SC)])
    o8 = c * _NP8 + r8base
    pltpu.sync_copy(acc_e.at[pl.ds(r8base, _SC)], eba)
    pltpu.sync_copy(eba, ep_out.at[pl.ds(o8, _SC)])
    pltpu.sync_copy(acc_e.at[pl.ds(r8base + _SC, _RP8 - _SC)],
                    eba.at[pl.ds(0, _RP8 - _SC)])
    pltpu.sync_copy(eba.at[pl.ds(0, _RP8 - _SC)],
                    ep_out.at[pl.ds(o8 + _SC, _RP8 - _SC)])


def _sc_scatter(numer, exps, edst2, zn):
    mesh = plsc.VectorSubcoreMesh(core_axis_name="c", subcore_axis_name="s",
                                  num_cores=_NC, num_subcores=_NS)
    buf = lambda: pltpu.VMEM((_SC, _D), _F32)
    f = pl.kernel(
        _scatter_body,
        out_type=[
            jax.ShapeDtypeStruct((_NC * _NP, _D), _F32),
            jax.ShapeDtypeStruct((_NC * _NP8, _D), _F32),
        ],
        mesh=mesh,
        scratch_types=[
            pltpu.VMEM_SHARED((_NP, _D), _F32),
            pltpu.VMEM_SHARED((_NP8, _D), _F32),
            pltpu.VMEM((1, _SC), jnp.int32),
            pltpu.VMEM((1, _SC), jnp.int32),
            pltpu.VMEM((1, _SC), jnp.int32),
            pltpu.VMEM((1, _SC), jnp.int32),
            buf(), buf(), buf(), buf(),
            pltpu.SemaphoreType.DMA,
            pltpu.SemaphoreType.DMA,
        ],
    )
    return f(numer, exps, edst2, zn)


def _final_body(n0, n1, e0, e1, wout, rmat, o):
    numer = n0[...] + n1[...]
    den = jnp.dot(e0[...] + e1[...], rmat[...], preferred_element_type=_F32)
    o[...] = jnp.dot(numer / (den + 1e-16), wout[...], preferred_element_type=_F32)


def _final_stage(np2, ep2, wout, rmat):
    grid = (_N // _BF,)
    off = _NP // _BF
    return pl.pallas_call(
        _final_body,
        grid=grid,
        in_specs=[
            pl.BlockSpec((_BF, _D), lambda i: (i, 0)),
            pl.BlockSpec((_BF, _D), lambda i: (i + off, 0)),
            pl.BlockSpec((_BF, 16), lambda i: (i, 0)),
            pl.BlockSpec((_BF, 16), lambda i: (i + off, 0)),
            pl.BlockSpec((_D, _D), lambda i: (0, 0)),
            pl.BlockSpec((16, _D), lambda i: (0, 0)),
        ],
        out_specs=pl.BlockSpec((_BF, _D), lambda i: (i, 0)),
        out_shape=jax.ShapeDtypeStruct((_N, _D), _F32),
    )(np2, np2, ep2, ep2, wout, rmat)


def kernel(node_feats, node_attr, edge_src, edge_dst, edge_attr, edge_embedding,
           W_src, b_src, W_dst, W1, b1, W2, b2, W3, b3,
           W_sph, W_alpha, W_lin, W_sph2, W_val, attn_dot, W_out):
    pad = _EP - _E
    esrc_p = jnp.concatenate([edge_src.astype(jnp.int32),
                              jnp.zeros((pad,), jnp.int32)])
    edst_p = jnp.concatenate([edge_dst.astype(jnp.int32),
                              jnp.zeros((pad,), jnp.int32)])
    ea16 = jnp.zeros((_EP, 16), _F32).at[:_E, :_DS].set(edge_attr)
    ee_p = jnp.zeros((_EP, _DE), _F32).at[:_E].set(edge_embedding)
    w1a, w1b, w1c = W1[:_DE], W1[_DE:_DE + _DA], W1[_DE + _DA:]
    wsph16 = jnp.zeros((16, _D), _F32).at[:_DS].set(W_sph)
    wsph2_16 = jnp.zeros((16, _D), _F32).at[:_DS].set(W_sph2)
    amat = (attn_dot[:, :, None] * jnp.eye(_H, dtype=_F32)[:, None, :])
    amat = amat.reshape(_H * _DH, _H)
    amat = jnp.pad(amat, ((0, 0), (0, 16 - _H)))
    rmat = (jnp.eye(_H, dtype=_F32)[:, :, None]
            * jnp.ones((_DH,), _F32)).reshape(_H, _H * _DH)
    rmat = jnp.pad(rmat, ((0, 16 - _H), (0, 0)))
    tmat = jnp.concatenate([jnp.eye(16, dtype=_F32)] * _H, axis=1)
    edst8_p = edst_p // 8
    oh = (jnp.arange(16, dtype=jnp.int32)[None, :]
          == (edst_p % 8)[:, None]).astype(_F32)
    zn = jnp.zeros((_NP, _D), _F32)

    t1, t2 = _node_proj(node_feats, node_attr, W_src,
                        b_src.reshape(1, _D), W_dst, w1b, w1c)
    t1e, t2e = _sc_gather(t1, t2,
                          esrc_p.reshape(_EP // _GC, _GC),
                          edst_p.reshape(_EP // _GC, _GC))
    numer, exps = _edge_stage(
        t1e, t2e, ee_p, ea16, oh,
        (w1a, b1.reshape(1, _FC), W2, b2.reshape(1, _FC),
         W3, b3.reshape(1, _D), wsph16, W_alpha, W_lin, wsph2_16, W_val,
         amat, rmat, tmat))
    np2, ep8 = _sc_scatter(numer, exps,
                           edst_p.reshape(_EP // _SC, _SC), zn)
    ep2 = ep8.reshape(_NC * _NP, 16)
    return _final_stage(np2, ep2, W_out, rmat)
